# baseline (speedup 1.0000x reference)
import functools

import jax
import jax.numpy as jnp
from jax import lax
from jax.experimental import pallas as pl
from jax.experimental.pallas import tpu as pltpu
from jax.experimental.pallas import tpu_sc as plsc

_RADII = (0.1, 0.2)
_NS = (16, 32)

_NC = 2
_NSUB = 16
_NW = _NC * _NSUB
_L = 16


_TDIMS = (((0,), (0,)), ((), ()))


def _rtne_bf16_bits(x):
    bits = lax.bitcast_convert_type(x, jnp.int32)
    lsb = jnp.bitwise_and(lax.shift_right_logical(bits, 16), 1)
    return jnp.bitwise_and(bits + 32767 + lsb, jnp.int32(-65536))


def _pack_u(u):
    lo = _rtne_bf16_bits(u[:, :64])
    hi = _rtne_bf16_bits(u[:, 64:])
    return jnp.bitwise_or(hi, lax.shift_right_logical(lo, 16))


def _unpack_u(rows):
    lo = lax.bitcast_convert_type(lax.shift_left(rows, 16), jnp.float32)
    hi = lax.bitcast_convert_type(jnp.bitwise_and(rows, jnp.int32(-65536)),
                                  jnp.float32)
    return lo, hi


def _u_kernel(f_ref, p_ref, wf0_ref, wx0_ref, b0_ref, wf1_ref, wx1_ref, b1_ref,
              u_ref):
    f = f_ref[0].astype(jnp.bfloat16)
    p = p_ref[...].astype(jnp.bfloat16)
    u0 = (lax.dot_general(f, wf0_ref[...], _TDIMS,
                          preferred_element_type=jnp.float32)
          + jnp.dot(p, wx0_ref[...], preferred_element_type=jnp.float32)
          + b0_ref[...])
    u1 = (lax.dot_general(f, wf1_ref[...], _TDIMS,
                          preferred_element_type=jnp.float32)
          + jnp.dot(p, wx1_ref[...], preferred_element_type=jnp.float32)
          + b1_ref[...])
    u_ref[...] = jnp.concatenate([_pack_u(u0), _pack_u(u1)], axis=1)


def _compute_u(features, xyz_rows, wf0, wx0, b0, wf1, wx1, b1):
    B = features.shape[0]
    N = features.shape[2]
    M = B * N
    NB = 2048
    grid = (B, N // NB)
    return pl.pallas_call(
        _u_kernel,
        grid=grid,
        in_specs=[
            pl.BlockSpec((1, 128, NB), lambda b, j: (b, 0, j)),
            pl.BlockSpec((NB, 3), lambda b, j: (b * (N // NB) + j, 0)),
            pl.BlockSpec((128, 128), lambda b, j: (0, 0)),
            pl.BlockSpec((3, 128), lambda b, j: (0, 0)),
            pl.BlockSpec((1, 128), lambda b, j: (0, 0)),
            pl.BlockSpec((128, 128), lambda b, j: (0, 0)),
            pl.BlockSpec((3, 128), lambda b, j: (0, 0)),
            pl.BlockSpec((1, 128), lambda b, j: (0, 0)),
        ],
        out_specs=pl.BlockSpec((NB, 128), lambda b, j: (b * (N // NB) + j, 0)),
        out_shape=jax.ShapeDtypeStruct((M, 128), jnp.int32),
        compiler_params=pltpu.CompilerParams(fuse_transposed_lhs_in_matmul=True),
    )(features, xyz_rows, wf0, wx0, b0, wf1, wx1, b1)


_K = 8


def _ballquery_kernel(B, N, S, C_PER_W, xyzh, centh,
                      idx0_hbm, idx1_hbm, xyzv, p2v, centv, buf0, buf1):
    w = lax.axis_index("s") * _NC + lax.axis_index("c")
    subs_per_b = _NW // B
    b = w // subs_per_b
    cbase = (w % subs_per_b) * C_PER_W

    pltpu.sync_copy(xyzh.at[pl.ds(b * N * 3, N * 3)], xyzv)
    pltpu.sync_copy(centh.at[pl.ds((b * S + cbase) * 3, C_PER_W * 3)], centv)

    lane = lax.iota(jnp.int32, _L)
    r0sq = jnp.float32(_RADII[0] * _RADII[0])
    r1sq = jnp.float32(_RADII[1] * _RADII[1])
    row_base = b * N

    def bf16_round(v):
        bits = plsc.bitcast(v, jnp.int32)
        lsb = jnp.bitwise_and(jnp.right_shift(bits, 16), 1)
        r = jnp.bitwise_and(bits + 32767 + lsb, jnp.int32(-65536))
        return plsc.bitcast(r, jnp.float32)

    def init_body(ic, _):
        pidx = ic * _L + lane
        b3 = pidx * 3
        px = plsc.load_gather(xyzv, [b3])
        py = plsc.load_gather(xyzv, [b3 + 1])
        pz = plsc.load_gather(xyzv, [b3 + 2])
        plsc.store_scatter(p2v, [pidx], (px * px + py * py) + pz * pz)
        plsc.store_scatter(xyzv, [b3], bf16_round(px))
        plsc.store_scatter(xyzv, [b3 + 1], bf16_round(py))
        plsc.store_scatter(xyzv, [b3 + 2], bf16_round(pz))
        return 0

    lax.fori_loop(0, N // _L, init_body, 0)

    zero_v = jnp.zeros((_L,), jnp.int32)

    def center_body(i, _):
        i3 = jnp.full((_L,), 3 * i, jnp.int32)
        cxo = plsc.load_gather(centv, [i3])
        cyo = plsc.load_gather(centv, [i3 + 1])
        czo = plsc.load_gather(centv, [i3 + 2])
        q2 = (cxo * cxo + cyo * cyo) + czo * czo
        cx = bf16_round(cxo)
        cy = bf16_round(cyo)
        cz = bf16_round(czo)

        def cond(carry):
            n, c0v, c1v = carry
            return (n < N) & jnp.any((c0v < _NS[0]) | (c1v < _NS[1]))

        def body(carry):
            n, c0v, c1v = carry
            m0s, m1s, gidxs = [], [], []
            for k in range(_K):
                pidx = n + k * _L + lane
                b3 = pidx * 3
                px = plsc.load_gather(xyzv, [b3])
                py = plsc.load_gather(xyzv, [b3 + 1])
                pz = plsc.load_gather(xyzv, [b3 + 2])
                p2 = plsc.load_gather(p2v, [pidx])
                qp = (cx * px + cy * py) + cz * pz
                d2 = (q2 + p2) - 2.0 * qp
                m0s.append(d2 <= r0sq)
                m1s.append(d2 <= r1sq)
                gidxs.append(pidx + row_base)
            t0s = [plsc.all_reduce_population_count(m) for m in m0s]
            t1s = [plsc.all_reduce_population_count(m) for m in m1s]

            def tree(xs, op):
                xs = list(xs)
                while len(xs) > 1:
                    xs = [op(xs[j], xs[j + 1]) if j + 1 < len(xs) else xs[j]
                          for j in range(0, len(xs), 2)]
                return xs[0]

            or0 = tree(m0s, lambda a, b: a | b)
            or1 = tree(m1s, lambda a, b: a | b)
            g0 = jnp.any(or0 & (c0v < _NS[0]))
            g1 = jnp.any(or1 & (c1v < _NS[1]))

            @pl.when(g1)
            def _():
                off = c1v
                for k in range(_K):
                    mi = m1s[k].astype(jnp.int32)
                    pos = off + plsc.cumsum(mi) - mi
                    plsc.store_scatter(buf1, [i * _NS[1] + pos], gidxs[k],
                                       mask=m1s[k] & (pos < _NS[1]))
                    off = off + t1s[k]

            @pl.when(g0)
            def _():
                off = c0v
                for k in range(_K):
                    mi = m0s[k].astype(jnp.int32)
                    pos = off + plsc.cumsum(mi) - mi
                    plsc.store_scatter(buf0, [i * _NS[0] + pos], gidxs[k],
                                       mask=m0s[k] & (pos < _NS[0]))
                    off = off + t0s[k]

            c0v = c0v + tree(t0s, lambda a, b: a + b)
            c1v = c1v + tree(t1s, lambda a, b: a + b)
            return (n + _K * _L, c0v, c1v)

        _, c0v, c1v = lax.while_loop(cond, body, (jnp.int32(0), zero_v, zero_v))

        empty_fill = jnp.full((_L,), row_base + N - 1, jnp.int32)
        first0 = plsc.load_gather(buf0, [jnp.full((_L,), i * _NS[0], jnp.int32)])
        fill0 = jnp.where(c0v > 0, first0, empty_fill)
        slot0 = i * _NS[0] + lane
        cur0 = plsc.load_gather(buf0, [slot0])
        plsc.store_scatter(buf0, [slot0], jnp.where(lane < c0v, cur0, fill0))

        first1 = plsc.load_gather(buf1, [jnp.full((_L,), i * _NS[1], jnp.int32)])
        fill1 = jnp.where(c1v > 0, first1, empty_fill)
        for j in range(_NS[1] // _L):
            slot1 = i * _NS[1] + j * _L + lane
            cur1 = plsc.load_gather(buf1, [slot1])
            plsc.store_scatter(buf1, [slot1],
                               jnp.where(j * _L + lane < c1v, cur1, fill1))
        return 0

    lax.fori_loop(0, C_PER_W, center_body, 0)

    n0 = C_PER_W * _NS[0]
    n1 = C_PER_W * _NS[1]
    pltpu.sync_copy(buf0, idx0_hbm.at[pl.ds(w * n0, n0)])
    pltpu.sync_copy(buf1, idx1_hbm.at[pl.ds(w * n1, n1)])


def _ball_query_sc(B, N, S, xyzh, centh):
    C_PER_W = (B * S) // _NW
    mesh = plsc.VectorSubcoreMesh(core_axis_name="c", subcore_axis_name="s")
    return pl.kernel(
        functools.partial(_ballquery_kernel, B, N, S, C_PER_W),
        out_type=[
            jax.ShapeDtypeStruct((B * S * _NS[0],), jnp.int32),
            jax.ShapeDtypeStruct((B * S * _NS[1],), jnp.int32),
        ],
        mesh=mesh,
        compiler_params=pltpu.CompilerParams(needs_layout_passes=False),
        scratch_types=[
            pltpu.VMEM((N * 3,), jnp.float32),
            pltpu.VMEM((N,), jnp.float32),
            pltpu.VMEM((C_PER_W * 3,), jnp.float32),
            pltpu.VMEM((C_PER_W * _NS[0],), jnp.int32),
            pltpu.VMEM((C_PER_W * _NS[1],), jnp.int32),
        ],
    )(xyzh, centh)


def _gather_kernel(CH, n0_per_w, n1_per_w, u_hbm, idx0_hbm, idx1_hbm,
                   r0_hbm, r1_hbm, idxv0, idxv1, rows_a, rows_b, sem_a, sem_b):
    w = lax.axis_index("s") * _NC + lax.axis_index("c")
    base0 = w * n0_per_w
    base1 = w * n1_per_w

    pltpu.sync_copy(idx0_hbm.at[pl.ds(base0, n0_per_w)], idxv0)
    pltpu.sync_copy(idx1_hbm.at[pl.ds(base1, n1_per_w)], idxv1)

    chunks = ([(u_hbm, idxv0, r0_hbm, base0, c) for c in range(n0_per_w // CH)]
              + [(u_hbm, idxv1, r1_hbm, base1, c) for c in range(n1_per_w // CH)])
    bufs = [(rows_a, sem_a), (rows_b, sem_b)]
    handles = [None, None]
    for j, (u, iv, r, base, c) in enumerate(chunks):
        rows, sem = bufs[j % 2]
        if handles[j % 2] is not None:
            handles[j % 2] = None
        copy = pltpu.async_copy(u.at[iv.at[pl.ds(c * CH, CH)]], rows, sem)
        if j >= 1:
            pu, piv, pr, pbase, pc = chunks[j - 1]
            prows, psem = bufs[(j - 1) % 2]
            handles[(j - 1) % 2].wait()
            pltpu.sync_copy(prows, pr.at[pl.ds(pbase + pc * CH, CH)])
        handles[j % 2] = copy
    lu, liv, lr, lbase, lc = chunks[-1]
    lrows, _ = bufs[(len(chunks) - 1) % 2]
    handles[(len(chunks) - 1) % 2].wait()
    pltpu.sync_copy(lrows, lr.at[pl.ds(lbase + lc * CH, CH)])


def _gather_rows_sc(u, idx0, idx1):
    M0 = idx0.shape[0]
    M1 = idx1.shape[0]
    CH = 128
    n0_per_w = M0 // _NW
    n1_per_w = M1 // _NW
    mesh = plsc.VectorSubcoreMesh(core_axis_name="c", subcore_axis_name="s")
    return pl.kernel(
        functools.partial(_gather_kernel, CH, n0_per_w, n1_per_w),
        out_type=[
            jax.ShapeDtypeStruct((M0, 128), jnp.int32),
            jax.ShapeDtypeStruct((M1, 128), jnp.int32),
        ],
        mesh=mesh,
        compiler_params=pltpu.CompilerParams(needs_layout_passes=False),
        scratch_types=[
            pltpu.VMEM((n0_per_w,), jnp.int32),
            pltpu.VMEM((n1_per_w,), jnp.int32),
            pltpu.VMEM((CH, 128), jnp.int32),
            pltpu.VMEM((CH, 128), jnp.int32),
            pltpu.SemaphoreType.DMA,
            pltpu.SemaphoreType.DMA,
        ],
    )(u, idx0, idx1)


def _head_kernel(ns, CB, col, rows_ref, cent_ref, wx_ref, w2_ref, b2_ref, out_ref):
    ulo, uhi = _unpack_u(rows_ref[:, col * 64:(col + 1) * 64])
    v = jnp.dot(cent_ref[...], wx_ref[...],
                preferred_element_type=jnp.float32)
    h1lo = jnp.maximum(ulo.reshape(CB, ns, 64) - v[:, None, :64], 0.0)
    h1hi = jnp.maximum(uhi.reshape(CB, ns, 64) - v[:, None, 64:], 0.0)
    w2 = w2_ref[...]
    h2 = (jnp.dot(h1lo.reshape(CB * ns, 64).astype(jnp.bfloat16), w2[:64],
                  preferred_element_type=jnp.float32)
          + jnp.dot(h1hi.reshape(CB * ns, 64).astype(jnp.bfloat16), w2[64:],
                    preferred_element_type=jnp.float32)
          + b2_ref[...])
    h2 = jnp.maximum(h2, 0.0)
    O = h2.shape[-1]
    out_ref[...] = jnp.max(h2.reshape(CB, ns, O), axis=1)


def _head(rows, cent_rows, wx, w2t, b2, ns, col):
    M = cent_rows.shape[0]
    O = w2t.shape[1]
    CB = 32
    return pl.pallas_call(
        functools.partial(_head_kernel, ns, CB, col),
        grid=(M // CB,),
        in_specs=[
            pl.BlockSpec((CB * ns, 128), lambda i: (i, 0)),
            pl.BlockSpec((CB, 3), lambda i: (i, 0)),
            pl.BlockSpec((3, 128), lambda i: (0, 0)),
            pl.BlockSpec((128, O), lambda i: (0, 0)),
            pl.BlockSpec((1, O), lambda i: (0, 0)),
        ],
        out_specs=pl.BlockSpec((CB, O), lambda i: (i, 0)),
        out_shape=jax.ShapeDtypeStruct((M, O), jnp.float32),
    )(rows, cent_rows, wx, w2t, b2)


def kernel(xyz, features, new_xyz, W00, b00, W01, b01, W10, b10, W11, b11):
    B, N, _ = xyz.shape
    S = new_xyz.shape[1]

    xyz_rows = xyz.reshape(B * N, 3)
    cent_rows = new_xyz.reshape(B * S, 3)
    xyzh = xyz.reshape(B * N * 3)
    centh = new_xyz.reshape(B * S * 3)

    wx0 = W00[:, :3].T
    wf0 = W00[:, 3:].T.astype(jnp.bfloat16)
    wx1 = W10[:, :3].T
    wf1 = W10[:, 3:].T.astype(jnp.bfloat16)

    u = _compute_u(features, xyz_rows, wf0, wx0.astype(jnp.bfloat16),
                   b00[None, :], wf1, wx1.astype(jnp.bfloat16),
                   b10[None, :])
    idx0, idx1 = _ball_query_sc(B, N, S, xyzh, centh)
    r0, r1 = _gather_rows_sc(u, idx0, idx1)

    y0 = _head(r0, cent_rows, wx0, W01.T.astype(jnp.bfloat16),
               b01[None, :], _NS[0], 0)
    y1 = _head(r1, cent_rows, wx1, W11.T.astype(jnp.bfloat16),
               b11[None, :], _NS[1], 1)

    y = jnp.concatenate([y0.reshape(B, S, 128), y1.reshape(B, S, 256)], axis=-1)
    return jnp.transpose(y, (0, 2, 1))


# speedup vs baseline: 1.0314x; 1.0314x over previous
import functools

import jax
import jax.numpy as jnp
from jax import lax
from jax.experimental import pallas as pl
from jax.experimental.pallas import tpu as pltpu
from jax.experimental.pallas import tpu_sc as plsc

_RADII = (0.1, 0.2)
_NS = (16, 32)

_NC = 2
_NSUB = 16
_NW = _NC * _NSUB
_L = 16


_TDIMS = (((0,), (0,)), ((), ()))


def _rtne_bf16_bits(x):
    bits = lax.bitcast_convert_type(x, jnp.int32)
    lsb = jnp.bitwise_and(lax.shift_right_logical(bits, 16), 1)
    return jnp.bitwise_and(bits + 32767 + lsb, jnp.int32(-65536))


def _pack_u(u):
    lo = _rtne_bf16_bits(u[:, :64])
    hi = _rtne_bf16_bits(u[:, 64:])
    return jnp.bitwise_or(hi, lax.shift_right_logical(lo, 16))


def _unpack_u(rows):
    lo = lax.bitcast_convert_type(lax.shift_left(rows, 16), jnp.float32)
    hi = lax.bitcast_convert_type(jnp.bitwise_and(rows, jnp.int32(-65536)),
                                  jnp.float32)
    return lo, hi


def _u_kernel(f_ref, p_ref, wf0_ref, wx0_ref, b0_ref, wf1_ref, wx1_ref, b1_ref,
              u0_ref, u1_ref):
    f = f_ref[0].astype(jnp.bfloat16)
    p = p_ref[...].astype(jnp.bfloat16)
    u0_ref[...] = (lax.dot_general(f, wf0_ref[...], _TDIMS,
                                   preferred_element_type=jnp.float32)
                   + jnp.dot(p, wx0_ref[...], preferred_element_type=jnp.float32)
                   + b0_ref[...])
    u1_ref[...] = (lax.dot_general(f, wf1_ref[...], _TDIMS,
                                   preferred_element_type=jnp.float32)
                   + jnp.dot(p, wx1_ref[...], preferred_element_type=jnp.float32)
                   + b1_ref[...])


def _compute_u(features, xyz_rows, wf0, wx0, b0, wf1, wx1, b1):
    B = features.shape[0]
    N = features.shape[2]
    M = B * N
    NB = 2048
    grid = (B, N // NB)
    return pl.pallas_call(
        _u_kernel,
        grid=grid,
        in_specs=[
            pl.BlockSpec((1, 128, NB), lambda b, j: (b, 0, j)),
            pl.BlockSpec((NB, 3), lambda b, j: (b * (N // NB) + j, 0)),
            pl.BlockSpec((128, 128), lambda b, j: (0, 0)),
            pl.BlockSpec((3, 128), lambda b, j: (0, 0)),
            pl.BlockSpec((1, 128), lambda b, j: (0, 0)),
            pl.BlockSpec((128, 128), lambda b, j: (0, 0)),
            pl.BlockSpec((3, 128), lambda b, j: (0, 0)),
            pl.BlockSpec((1, 128), lambda b, j: (0, 0)),
        ],
        out_specs=[
            pl.BlockSpec((NB, 128), lambda b, j: (b * (N // NB) + j, 0)),
            pl.BlockSpec((NB, 128), lambda b, j: (b * (N // NB) + j, 0)),
        ],
        out_shape=[
            jax.ShapeDtypeStruct((M, 128), jnp.float32),
            jax.ShapeDtypeStruct((M, 128), jnp.float32),
        ],
        compiler_params=pltpu.CompilerParams(fuse_transposed_lhs_in_matmul=True),
    )(features, xyz_rows, wf0, wx0, b0, wf1, wx1, b1)


_K = 8


def _ballquery_kernel(B, N, S, C_PER_W, xyzh, centh,
                      idx0_hbm, idx1_hbm, xyzv, p2v, centv, buf0, buf1):
    w = lax.axis_index("s") * _NC + lax.axis_index("c")
    subs_per_b = _NW // B
    b = w // subs_per_b
    cbase = (w % subs_per_b) * C_PER_W

    pltpu.sync_copy(xyzh.at[pl.ds(b * N * 3, N * 3)], xyzv)
    pltpu.sync_copy(centh.at[pl.ds((b * S + cbase) * 3, C_PER_W * 3)], centv)

    lane = lax.iota(jnp.int32, _L)
    r0sq = jnp.float32(_RADII[0] * _RADII[0])
    r1sq = jnp.float32(_RADII[1] * _RADII[1])
    row_base = b * N

    def bf16_round(v):
        bits = plsc.bitcast(v, jnp.int32)
        lsb = jnp.bitwise_and(jnp.right_shift(bits, 16), 1)
        r = jnp.bitwise_and(bits + 32767 + lsb, jnp.int32(-65536))
        return plsc.bitcast(r, jnp.float32)

    def init_body(ic, _):
        pidx = ic * _L + lane
        b3 = pidx * 3
        px = plsc.load_gather(xyzv, [b3])
        py = plsc.load_gather(xyzv, [b3 + 1])
        pz = plsc.load_gather(xyzv, [b3 + 2])
        plsc.store_scatter(p2v, [pidx], (px * px + py * py) + pz * pz)
        plsc.store_scatter(xyzv, [b3], bf16_round(px))
        plsc.store_scatter(xyzv, [b3 + 1], bf16_round(py))
        plsc.store_scatter(xyzv, [b3 + 2], bf16_round(pz))
        return 0

    lax.fori_loop(0, N // _L, init_body, 0)

    zero_v = jnp.zeros((_L,), jnp.int32)

    def center_body(i, _):
        i3 = jnp.full((_L,), 3 * i, jnp.int32)
        cxo = plsc.load_gather(centv, [i3])
        cyo = plsc.load_gather(centv, [i3 + 1])
        czo = plsc.load_gather(centv, [i3 + 2])
        q2 = (cxo * cxo + cyo * cyo) + czo * czo
        cx = bf16_round(cxo)
        cy = bf16_round(cyo)
        cz = bf16_round(czo)

        def cond(carry):
            n, c0v, c1v = carry
            return (n < N) & jnp.any((c0v < _NS[0]) | (c1v < _NS[1]))

        def body(carry):
            n, c0v, c1v = carry
            m0s, m1s, gidxs = [], [], []
            for k in range(_K):
                pidx = n + k * _L + lane
                b3 = pidx * 3
                px = plsc.load_gather(xyzv, [b3])
                py = plsc.load_gather(xyzv, [b3 + 1])
                pz = plsc.load_gather(xyzv, [b3 + 2])
                p2 = plsc.load_gather(p2v, [pidx])
                qp = (cx * px + cy * py) + cz * pz
                d2 = (q2 + p2) - 2.0 * qp
                m0s.append(d2 <= r0sq)
                m1s.append(d2 <= r1sq)
                gidxs.append(pidx + row_base)
            t0s = [plsc.all_reduce_population_count(m) for m in m0s]
            t1s = [plsc.all_reduce_population_count(m) for m in m1s]

            def tree(xs, op):
                xs = list(xs)
                while len(xs) > 1:
                    xs = [op(xs[j], xs[j + 1]) if j + 1 < len(xs) else xs[j]
                          for j in range(0, len(xs), 2)]
                return xs[0]

            or0 = tree(m0s, lambda a, b: a | b)
            or1 = tree(m1s, lambda a, b: a | b)
            g0 = jnp.any(or0 & (c0v < _NS[0]))
            g1 = jnp.any(or1 & (c1v < _NS[1]))

            @pl.when(g1)
            def _():
                off = c1v
                for k in range(_K):
                    mi = m1s[k].astype(jnp.int32)
                    pos = off + plsc.cumsum(mi) - mi
                    plsc.store_scatter(buf1, [i * _NS[1] + pos], gidxs[k],
                                       mask=m1s[k] & (pos < _NS[1]))
                    off = off + t1s[k]

            @pl.when(g0)
            def _():
                off = c0v
                for k in range(_K):
                    mi = m0s[k].astype(jnp.int32)
                    pos = off + plsc.cumsum(mi) - mi
                    plsc.store_scatter(buf0, [i * _NS[0] + pos], gidxs[k],
                                       mask=m0s[k] & (pos < _NS[0]))
                    off = off + t0s[k]

            c0v = c0v + tree(t0s, lambda a, b: a + b)
            c1v = c1v + tree(t1s, lambda a, b: a + b)
            return (n + _K * _L, c0v, c1v)

        _, c0v, c1v = lax.while_loop(cond, body, (jnp.int32(0), zero_v, zero_v))

        empty_fill = jnp.full((_L,), row_base + N - 1, jnp.int32)
        first0 = plsc.load_gather(buf0, [jnp.full((_L,), i * _NS[0], jnp.int32)])
        fill0 = jnp.where(c0v > 0, first0, empty_fill)
        slot0 = i * _NS[0] + lane
        cur0 = plsc.load_gather(buf0, [slot0])
        plsc.store_scatter(buf0, [slot0], jnp.where(lane < c0v, cur0, fill0))

        first1 = plsc.load_gather(buf1, [jnp.full((_L,), i * _NS[1], jnp.int32)])
        fill1 = jnp.where(c1v > 0, first1, empty_fill)
        for j in range(_NS[1] // _L):
            slot1 = i * _NS[1] + j * _L + lane
            cur1 = plsc.load_gather(buf1, [slot1])
            plsc.store_scatter(buf1, [slot1],
                               jnp.where(j * _L + lane < c1v, cur1, fill1))
        return 0

    lax.fori_loop(0, C_PER_W, center_body, 0)

    n0 = C_PER_W * _NS[0]
    n1 = C_PER_W * _NS[1]
    pltpu.sync_copy(buf0, idx0_hbm.at[pl.ds(w * n0, n0)])
    pltpu.sync_copy(buf1, idx1_hbm.at[pl.ds(w * n1, n1)])


def _ball_query_sc(B, N, S, xyzh, centh):
    C_PER_W = (B * S) // _NW
    mesh = plsc.VectorSubcoreMesh(core_axis_name="c", subcore_axis_name="s")
    return pl.kernel(
        functools.partial(_ballquery_kernel, B, N, S, C_PER_W),
        out_type=[
            jax.ShapeDtypeStruct((B * S * _NS[0],), jnp.int32),
            jax.ShapeDtypeStruct((B * S * _NS[1],), jnp.int32),
        ],
        mesh=mesh,
        compiler_params=pltpu.CompilerParams(needs_layout_passes=False),
        scratch_types=[
            pltpu.VMEM((N * 3,), jnp.float32),
            pltpu.VMEM((N,), jnp.float32),
            pltpu.VMEM((C_PER_W * 3,), jnp.float32),
            pltpu.VMEM((C_PER_W * _NS[0],), jnp.int32),
            pltpu.VMEM((C_PER_W * _NS[1],), jnp.int32),
        ],
    )(xyzh, centh)


def _gather_kernel(CH, n0_per_w, n1_per_w, u0_hbm, u1_hbm, idx0_hbm, idx1_hbm,
                   r0_hbm, r1_hbm, idxv0, idxv1, rows_a, rows_b, sem_a, sem_b):
    w = lax.axis_index("s") * _NC + lax.axis_index("c")
    base0 = w * n0_per_w
    base1 = w * n1_per_w

    pltpu.sync_copy(idx0_hbm.at[pl.ds(base0, n0_per_w)], idxv0)
    pltpu.sync_copy(idx1_hbm.at[pl.ds(base1, n1_per_w)], idxv1)

    chunks = ([(u0_hbm, idxv0, r0_hbm, base0, c) for c in range(n0_per_w // CH)]
              + [(u1_hbm, idxv1, r1_hbm, base1, c) for c in range(n1_per_w // CH)])
    bufs = [(rows_a, sem_a), (rows_b, sem_b)]
    handles = [None, None]
    for j, (u, iv, r, base, c) in enumerate(chunks):
        rows, sem = bufs[j % 2]
        if handles[j % 2] is not None:
            handles[j % 2] = None
        copy = pltpu.async_copy(u.at[iv.at[pl.ds(c * CH, CH)]], rows, sem)
        if j >= 1:
            pu, piv, pr, pbase, pc = chunks[j - 1]
            prows, psem = bufs[(j - 1) % 2]
            handles[(j - 1) % 2].wait()
            pltpu.sync_copy(prows, pr.at[pl.ds(pbase + pc * CH, CH)])
        handles[j % 2] = copy
    lu, liv, lr, lbase, lc = chunks[-1]
    lrows, _ = bufs[(len(chunks) - 1) % 2]
    handles[(len(chunks) - 1) % 2].wait()
    pltpu.sync_copy(lrows, lr.at[pl.ds(lbase + lc * CH, CH)])


def _gather_rows_sc(u0, u1, idx0, idx1):
    M0 = idx0.shape[0]
    M1 = idx1.shape[0]
    CH = 128
    n0_per_w = M0 // _NW
    n1_per_w = M1 // _NW
    mesh = plsc.VectorSubcoreMesh(core_axis_name="c", subcore_axis_name="s")
    return pl.kernel(
        functools.partial(_gather_kernel, CH, n0_per_w, n1_per_w),
        out_type=[
            jax.ShapeDtypeStruct((M0, 128), jnp.float32),
            jax.ShapeDtypeStruct((M1, 128), jnp.float32),
        ],
        mesh=mesh,
        compiler_params=pltpu.CompilerParams(needs_layout_passes=False),
        scratch_types=[
            pltpu.VMEM((n0_per_w,), jnp.int32),
            pltpu.VMEM((n1_per_w,), jnp.int32),
            pltpu.VMEM((CH, 128), jnp.float32),
            pltpu.VMEM((CH, 128), jnp.float32),
            pltpu.SemaphoreType.DMA,
            pltpu.SemaphoreType.DMA,
        ],
    )(u0, u1, idx0, idx1)


def _head_kernel(ns, CB, rows_ref, cent_ref, wx_ref, w2_ref, b2_ref, out_ref):
    rows = rows_ref[...]
    v = jnp.dot(cent_ref[...], wx_ref[...],
                preferred_element_type=jnp.float32)
    h1 = jnp.maximum(rows.reshape(CB, ns, 128) - v[:, None, :], 0.0)
    h2 = jnp.dot(h1.reshape(CB * ns, 128).astype(jnp.bfloat16), w2_ref[...],
                 preferred_element_type=jnp.float32) + b2_ref[...]
    h2 = jnp.maximum(h2, 0.0)
    O = h2.shape[-1]
    out_ref[...] = jnp.max(h2.reshape(CB, ns, O), axis=1)


def _head(rows, cent_rows, wx, w2t, b2, ns):
    M = cent_rows.shape[0]
    O = w2t.shape[1]
    CB = 32
    return pl.pallas_call(
        functools.partial(_head_kernel, ns, CB),
        grid=(M // CB,),
        in_specs=[
            pl.BlockSpec((CB * ns, 128), lambda i: (i, 0)),
            pl.BlockSpec((CB, 3), lambda i: (i, 0)),
            pl.BlockSpec((3, 128), lambda i: (0, 0)),
            pl.BlockSpec((128, O), lambda i: (0, 0)),
            pl.BlockSpec((1, O), lambda i: (0, 0)),
        ],
        out_specs=pl.BlockSpec((CB, O), lambda i: (i, 0)),
        out_shape=jax.ShapeDtypeStruct((M, O), jnp.float32),
    )(rows, cent_rows, wx, w2t, b2)


def kernel(xyz, features, new_xyz, W00, b00, W01, b01, W10, b10, W11, b11):
    B, N, _ = xyz.shape
    S = new_xyz.shape[1]

    xyz_rows = xyz.reshape(B * N, 3)
    cent_rows = new_xyz.reshape(B * S, 3)
    xyzh = xyz.reshape(B * N * 3)
    centh = new_xyz.reshape(B * S * 3)

    wx0 = W00[:, :3].T
    wf0 = W00[:, 3:].T.astype(jnp.bfloat16)
    wx1 = W10[:, :3].T
    wf1 = W10[:, 3:].T.astype(jnp.bfloat16)

    u0, u1 = _compute_u(features, xyz_rows, wf0, wx0.astype(jnp.bfloat16),
                        b00[None, :], wf1, wx1.astype(jnp.bfloat16),
                        b10[None, :])
    idx0, idx1 = _ball_query_sc(B, N, S, xyzh, centh)
    r0, r1 = _gather_rows_sc(u0, u1, idx0, idx1)

    y0 = _head(r0, cent_rows, wx0, W01.T.astype(jnp.bfloat16),
               b01[None, :], _NS[0])
    y1 = _head(r1, cent_rows, wx1, W11.T.astype(jnp.bfloat16),
               b11[None, :], _NS[1])

    y = jnp.concatenate([y0.reshape(B, S, 128), y1.reshape(B, S, 256)], axis=-1)
    return jnp.transpose(y, (0, 2, 1))
